# Initial kernel scaffold; baseline (speedup 1.0000x reference)
import jax
import jax.numpy as jnp
from jax.experimental import pallas as pl


def kernel(node, pair, l1_feats, pair_index, edge_src, edge_dst, edge_sh, proj_l0_w, proj_l0_b, ln_g, ln_b, fc1_w, fc1_b, fc2_w, fc2_b, proj_node_w, proj_node_b):
    raise NotImplementedError("write your pallas kernel here")


# speedup vs baseline: 4.2573x; 4.2573x over previous
import functools

import jax
import jax.numpy as jnp
import numpy as np
from jax import lax
from jax.experimental import pallas as pl
from jax.experimental.pallas import tpu as pltpu
from jax.experimental.pallas import tpu_sc as plsc

L = 512
E = 16384
D_NODE = 256
D_PAIR = 128
L0 = 32
L1 = 8
BE = 1024
G = E // BE
NW = 32
EPW = E // NW
GC = 4
CW = EPW // GC

_N0 = 1.0 / np.sqrt(40.0)
_N1 = np.sqrt(3.0 / 48.0)
_I3 = 1.0 / np.sqrt(3.0)


def _np_c121():
    C = np.zeros((3, 3, 5), dtype=np.float32)
    c = 1.0 / np.sqrt(10.0)
    d = 1.0 / np.sqrt(30.0)
    C[0, 2, 0] = c; C[2, 0, 0] = c
    C[0, 1, 1] = c; C[1, 0, 1] = c
    C[1, 1, 2] = 2.0 * d; C[0, 0, 2] = -d; C[2, 2, 2] = -d
    C[1, 2, 3] = c; C[2, 1, 3] = c
    C[2, 2, 4] = c; C[0, 0, 4] = -c
    return np.transpose(C, (2, 0, 1)).reshape(5, 9)


def _f32(x):
    return jnp.asarray(x, dtype=jnp.float32)


_E = np.eye
_O = np.ones
_R_EXP_32_1024 = _f32(np.kron(_E(32), _O((1, 32))))
_R_FOLD_1024_32 = _f32(np.tile(_E(32), (32, 1)))
_R_EXP_32_256 = _f32(np.kron(_E(32), _O((1, 8))))
_R_FOLD_256_8 = _f32(np.tile(_E(8), (32, 1)))
_R_EXP_8_256 = _f32(np.kron(_E(8), _O((1, 32))))
_R_FOLD_256_32 = _f32(np.tile(_E(32), (8, 1)))
_R_M_24 = _f32(np.tile(_E(3), (1, 8)))
_R_FOLD_24_8 = _f32(np.kron(_E(8), _O((3, 1))))
_R_V_24 = _f32(np.kron(_E(8), _O((1, 3))))
_R_A_64_192 = _f32(np.kron(_E(64), _O((1, 3))))
_R_B_24_192 = _f32(np.kron(_E(8), np.tile(_E(3), (1, 8))))
_R_F_192_24 = _f32(np.tile(_E(24), (8, 1)))
_R_A_24_72 = _f32(np.kron(_E(24), _O((1, 3))))
_R_B_9_72 = _f32(np.tile(_E(9), (1, 8)))
_R_F_72_24 = _f32(np.kron(_E(8), np.tile(_E(3), (3, 1))))
_CD = _f32(_np_c121())


@functools.cache
def _build_sc_gather():
    @functools.partial(
        pl.kernel,
        mesh=plsc.VectorSubcoreMesh(core_axis_name="c", subcore_axis_name="s"),
        out_type=jax.ShapeDtypeStruct((E, D_PAIR), jnp.float32),
        scratch_types=[
            pltpu.VMEM((EPW,), jnp.int32),
            pltpu.VMEM((EPW,), jnp.int32),
            pltpu.VMEM((GC, CW), jnp.int32),
            pltpu.VMEM((GC, CW, D_PAIR), jnp.float32),
            pltpu.SemaphoreType.DMA,
        ],
    )
    def _sc_gather(ii_hbm, jj_hbm, table_hbm, out_hbm, ii_v, jj_v, idx_v,
                   rows_v, sem):
        wid = lax.axis_index("s") * 2 + lax.axis_index("c")
        base = wid * EPW
        pltpu.sync_copy(ii_hbm.at[pl.ds(base, EPW)], ii_v)
        pltpu.sync_copy(jj_hbm.at[pl.ds(base, EPW)], jj_v)
        for c in range(GC):
            for k in range(CW // 16):
                s = pl.ds(c * CW + k * 16, 16)
                idx_v[c, pl.ds(k * 16, 16)] = ii_v[s] * L + jj_v[s]
        copies = [
            pltpu.async_copy(table_hbm.at[idx_v.at[c]], rows_v.at[c], sem)
            for c in range(GC)
        ]
        for cp in copies:
            cp.wait()
        for c in range(GC):
            pltpu.sync_copy(rows_v.at[c],
                            out_hbm.at[pl.ds(base + c * CW, CW)])

    return _sc_gather


def _tc_body(ef_ref, src_ref, dst_ref, sh_ref, node_ref, l1f_ref,
             plw_ref, plb_ref, lng_ref, lnb_ref, f1w_ref, f1b_ref,
             w1w_ref, w1b_ref, w2w_ref, w2b_ref, w3w_ref, w3b_ref,
             w4w_ref, w4b_ref, w5w_ref, w5b_ref, pnw_ref, pnb_ref,
             cd_ref, e32w_ref, f1024_ref, e328_ref, f2568_ref, e832_ref,
             f25632_ref, rm24_ref, rf248_ref, rv24_ref, ra64_ref, rb24_ref,
             rf192_ref, ra24_ref, rb9_ref, rf72_ref,
             nodeout_ref, l1o_ref,
             l0_s, acc0_s, acc1_s, cnt_s):
    g = pl.program_id(0)

    @pl.when(g == 0)
    def _init():
        l0_s[:] = jnp.dot(node_ref[:], plw_ref[:],
                          preferred_element_type=jnp.float32) + plb_ref[:]
        acc0_s[:] = jnp.zeros_like(acc0_s)
        acc1_s[:] = jnp.zeros_like(acc1_s)
        cnt_s[:] = jnp.zeros_like(cnt_s)

    ef = ef_ref[:]
    mu = jnp.mean(ef, axis=1, keepdims=True)
    xc = ef - mu
    var = jnp.mean(xc * xc, axis=1, keepdims=True)
    h = xc * lax.rsqrt(var + 1e-5) * lng_ref[:] + lnb_ref[:]
    h = jnp.maximum(jnp.dot(h, f1w_ref[:],
                            preferred_element_type=jnp.float32) + f1b_ref[:],
                    0.0)

    w1m = jnp.dot(h, w1w_ref[:], preferred_element_type=jnp.float32) + w1b_ref[:]
    w2m = jnp.dot(h, w2w_ref[:], preferred_element_type=jnp.float32) + w2b_ref[:]
    w3m = jnp.dot(h, w3w_ref[:], preferred_element_type=jnp.float32) + w3b_ref[:]
    w4m = jnp.dot(h, w4w_ref[:], preferred_element_type=jnp.float32) + w4b_ref[:]
    w5m = jnp.dot(h, w5w_ref[:], preferred_element_type=jnp.float32) + w5b_ref[:]

    dstv = dst_ref[0]
    iota_n = lax.broadcasted_iota(jnp.int32, (BE, L), 1)
    ohd = (dstv == iota_n).astype(jnp.float32)
    xs = jnp.dot(ohd, l0_s[:], preferred_element_type=jnp.float32)
    xv = jnp.dot(ohd, l1f_ref[:], preferred_element_type=jnp.float32)

    sh = sh_ref[:]
    x2s = sh[:, 0:1]
    x2v = sh[:, 1:4]
    x2t = sh[:, 4:9]

    def md(a, b):
        return jnp.dot(a, b, preferred_element_type=jnp.float32)

    xs2 = xs * x2s
    o0 = _N0 * md(w1m * md(xs2, e32w_ref[:]), f1024_ref[:])
    x2v24 = md(x2v, rm24_ref[:])
    dotp = md(xv * x2v24, rf248_ref[:])
    o0 = o0 + (_N0 * _I3) * md(w4m * md(dotp, e832_ref[:]), f25632_ref[:])
    t2 = md(w2m * md(xs, e328_ref[:]), f2568_ref[:])
    out1 = (_N1 * _I3) * (md(t2, rv24_ref[:]) * x2v24)
    xvs = xv * x2s
    out1 = out1 + (_N1 * _I3) * md(
        md(w3m, ra64_ref[:]) * md(xvs, rb24_ref[:]), rf192_ref[:])
    dmat = md(x2t, cd_ref[:])
    tmp = md(md(xv, ra24_ref[:]) * md(dmat, rb9_ref[:]), rf72_ref[:])
    out1 = out1 + _N1 * md(
        md(w5m, ra64_ref[:]) * md(tmp, rb24_ref[:]), rf192_ref[:])

    srow = src_ref[0]
    iota_t = lax.broadcasted_iota(jnp.int32, (L, BE), 0)
    ohsT = (srow == iota_t).astype(jnp.float32)
    acc0_s[:] = acc0_s[:] + md(ohsT, o0)
    acc1_s[:] = acc1_s[:] + md(ohsT, out1)
    cnt_s[:] = cnt_s[:] + md(ohsT, jnp.ones((BE, 1), jnp.float32))

    @pl.when(g == G - 1)
    def _finish():
        cnt = jnp.maximum(cnt_s[:], 1.0)
        m0 = acc0_s[:] / cnt
        m1 = acc1_s[:] / cnt
        nodeout_ref[:] = (jnp.dot(m0, pnw_ref[:],
                                  preferred_element_type=jnp.float32)
                          + pnb_ref[:] + node_ref[:])
        l1o_ref[:] = m1 + l1f_ref[:]


def _rep(shape):
    nd = len(shape)
    return pl.BlockSpec(shape, lambda g, _n=nd: (0,) * _n)


def _build_tc(interpret: bool = False):
    in_specs = [
        pl.BlockSpec((BE, D_PAIR), lambda g: (g, 0)),
        pl.BlockSpec((1, 1, BE), lambda g: (g, 0, 0)),
        pl.BlockSpec((1, BE, 1), lambda g: (g, 0, 0)),
        pl.BlockSpec((BE, 9), lambda g: (g, 0)),
        _rep((L, D_NODE)),
        _rep((L, 3 * L1)),
        _rep((D_NODE, L0)), _rep((1, L0)),
        _rep((1, D_PAIR)), _rep((1, D_PAIR)),
        _rep((D_PAIR, D_PAIR)), _rep((1, D_PAIR)),
        _rep((D_PAIR, 1024)), _rep((1, 1024)),
        _rep((D_PAIR, 256)), _rep((1, 256)),
        _rep((D_PAIR, 64)), _rep((1, 64)),
        _rep((D_PAIR, 256)), _rep((1, 256)),
        _rep((D_PAIR, 64)), _rep((1, 64)),
        _rep((L0, D_NODE)), _rep((1, D_NODE)),
        _rep((5, 9)),
        _rep((32, 1024)), _rep((1024, 32)),
        _rep((32, 256)), _rep((256, 8)),
        _rep((8, 256)), _rep((256, 32)),
        _rep((3, 24)), _rep((24, 8)), _rep((8, 24)),
        _rep((64, 192)), _rep((24, 192)), _rep((192, 24)),
        _rep((24, 72)), _rep((9, 72)), _rep((72, 24)),
    ]
    out_specs = [
        pl.BlockSpec((L, D_NODE), lambda g: (0, 0)),
        pl.BlockSpec((L, 3 * L1), lambda g: (0, 0)),
    ]
    return pl.pallas_call(
        _tc_body,
        grid=(G,),
        in_specs=in_specs,
        out_specs=out_specs,
        out_shape=[
            jax.ShapeDtypeStruct((L, D_NODE), jnp.float32),
            jax.ShapeDtypeStruct((L, 3 * L1), jnp.float32),
        ],
        scratch_shapes=[
            pltpu.VMEM((L, L0), jnp.float32),
            pltpu.VMEM((L, L0), jnp.float32),
            pltpu.VMEM((L, 3 * L1), jnp.float32),
            pltpu.VMEM((L, 1), jnp.float32),
        ],
        compiler_params=pltpu.CompilerParams(
            dimension_semantics=("arbitrary",)),
        interpret=interpret,
    )


def kernel(node, pair, l1_feats, pair_index, edge_src, edge_dst, edge_sh,
           proj_l0_w, proj_l0_b, ln_g, ln_b, fc1_w, fc1_b, fc2_w, fc2_b,
           proj_node_w, proj_node_b):
    table = pair.reshape(L * L, D_PAIR)
    ef = _build_sc_gather()(pair_index[1], pair_index[2], table)

    r1 = lambda a: a.reshape(1, -1)
    args = (
        ef,
        edge_src.reshape(G, 1, BE),
        edge_dst.reshape(G, BE, 1),
        edge_sh,
        node.reshape(L, D_NODE),
        l1_feats.reshape(L, 3 * L1),
        proj_l0_w, r1(proj_l0_b), r1(ln_g), r1(ln_b),
        fc1_w, r1(fc1_b),
        fc2_w[:, 0:1024], r1(fc2_b[0:1024]),
        fc2_w[:, 1024:1280], r1(fc2_b[1024:1280]),
        fc2_w[:, 1280:1344], r1(fc2_b[1280:1344]),
        fc2_w[:, 1344:1600], r1(fc2_b[1344:1600]),
        fc2_w[:, 1600:1664], r1(fc2_b[1600:1664]),
        proj_node_w, r1(proj_node_b),
        _CD,
        _R_EXP_32_1024, _R_FOLD_1024_32,
        _R_EXP_32_256, _R_FOLD_256_8,
        _R_EXP_8_256, _R_FOLD_256_32,
        _R_M_24, _R_FOLD_24_8, _R_V_24,
        _R_A_64_192, _R_B_24_192, _R_F_192_24,
        _R_A_24_72, _R_B_9_72, _R_F_72_24,
    )
    node_out, l1o = _build_tc()(*args)
    return (node_out.reshape(1, L, D_NODE), l1o.reshape(1, L, 3 * L1))


# speedup vs baseline: 4.4996x; 1.0569x over previous
import functools

import jax
import jax.numpy as jnp
import numpy as np
from jax import lax
from jax.experimental import pallas as pl
from jax.experimental.pallas import tpu as pltpu
from jax.experimental.pallas import tpu_sc as plsc

L = 512
E = 16384
D_NODE = 256
D_PAIR = 128
L0 = 32
L1 = 8
BE = 2048
G = E // BE
NW = 32
EPW = E // NW
GC = 4
CW = EPW // GC

_N0 = 1.0 / np.sqrt(40.0)
_N1 = np.sqrt(3.0 / 48.0)
_I3 = 1.0 / np.sqrt(3.0)


def _np_c121():
    C = np.zeros((3, 3, 5), dtype=np.float32)
    c = 1.0 / np.sqrt(10.0)
    d = 1.0 / np.sqrt(30.0)
    C[0, 2, 0] = c; C[2, 0, 0] = c
    C[0, 1, 1] = c; C[1, 0, 1] = c
    C[1, 1, 2] = 2.0 * d; C[0, 0, 2] = -d; C[2, 2, 2] = -d
    C[1, 2, 3] = c; C[2, 1, 3] = c
    C[2, 2, 4] = c; C[0, 0, 4] = -c
    return np.transpose(C, (2, 0, 1)).reshape(5, 9)


def _f32(x):
    return jnp.asarray(x, dtype=jnp.float32)


_E = np.eye
_O = np.ones
_R_EXP_32_1024 = _f32(np.kron(_E(32), _O((1, 32))))
_R_FOLD_1024_32 = _f32(np.tile(_E(32), (32, 1)))
_R_EXP_32_256 = _f32(np.kron(_E(32), _O((1, 8))))
_R_FOLD_256_8 = _f32(np.tile(_E(8), (32, 1)))
_R_EXP_8_256 = _f32(np.kron(_E(8), _O((1, 32))))
_R_FOLD_256_32 = _f32(np.tile(_E(32), (8, 1)))
_R_M_24 = _f32(np.tile(_E(3), (1, 8)))
_R_FOLD_24_8 = _f32(np.kron(_E(8), _O((3, 1))))
_R_V_24 = _f32(np.kron(_E(8), _O((1, 3))))
_R_A_64_192 = _f32(np.kron(_E(64), _O((1, 3))))
_R_B_24_192 = _f32(np.kron(_E(8), np.tile(_E(3), (1, 8))))
_R_F_192_24 = _f32(np.tile(_E(24), (8, 1)))
_R_A_24_72 = _f32(np.kron(_E(24), _O((1, 3))))
_R_B_9_72 = _f32(np.tile(_E(9), (1, 8)))
_R_F_72_24 = _f32(np.kron(_E(8), np.tile(_E(3), (3, 1))))
_CD = _f32(_np_c121())


@functools.cache
def _build_sc_gather():
    @functools.partial(
        pl.kernel,
        mesh=plsc.VectorSubcoreMesh(core_axis_name="c", subcore_axis_name="s"),
        out_type=jax.ShapeDtypeStruct((E, D_PAIR), jnp.float32),
        scratch_types=[
            pltpu.VMEM((EPW,), jnp.int32),
            pltpu.VMEM((EPW,), jnp.int32),
            pltpu.VMEM((GC, CW), jnp.int32),
            pltpu.VMEM((GC, CW, D_PAIR), jnp.float32),
            pltpu.SemaphoreType.DMA,
        ],
    )
    def _sc_gather(ii_hbm, jj_hbm, table_hbm, out_hbm, ii_v, jj_v, idx_v,
                   rows_v, sem):
        wid = lax.axis_index("s") * 2 + lax.axis_index("c")
        base = wid * EPW
        pltpu.sync_copy(ii_hbm.at[pl.ds(base, EPW)], ii_v)
        pltpu.sync_copy(jj_hbm.at[pl.ds(base, EPW)], jj_v)
        for c in range(GC):
            for k in range(CW // 16):
                s = pl.ds(c * CW + k * 16, 16)
                idx_v[c, pl.ds(k * 16, 16)] = ii_v[s] * L + jj_v[s]
        copies = [
            pltpu.async_copy(table_hbm.at[idx_v.at[c]], rows_v.at[c], sem)
            for c in range(GC)
        ]
        for cp in copies:
            cp.wait()
        for c in range(GC):
            pltpu.sync_copy(rows_v.at[c],
                            out_hbm.at[pl.ds(base + c * CW, CW)])

    return _sc_gather


def _tc_body(ef_ref, src_ref, dst_ref, sh_ref, node_ref, l1f_ref,
             plw_ref, plb_ref, lng_ref, lnb_ref, f1w_ref, f1b_ref,
             w1w_ref, w1b_ref, w2w_ref, w2b_ref, w3w_ref, w3b_ref,
             w4w_ref, w4b_ref, w5w_ref, w5b_ref, pnw_ref, pnb_ref,
             cd_ref, e32w_ref, f1024_ref, e328_ref, f2568_ref, e832_ref,
             f25632_ref, rm24_ref, rf248_ref, rv24_ref, ra64_ref, rb24_ref,
             rf192_ref, ra24_ref, rb9_ref, rf72_ref,
             nodeout_ref, l1o_ref,
             l0_s, acc0_s, acc1_s, cnt_s):
    g = pl.program_id(0)

    @pl.when(g == 0)
    def _init():
        l0_s[:] = jnp.dot(node_ref[:], plw_ref[:],
                          preferred_element_type=jnp.float32) + plb_ref[:]
        acc0_s[:] = jnp.zeros_like(acc0_s)
        acc1_s[:] = jnp.zeros_like(acc1_s)
        cnt_s[:] = jnp.zeros_like(cnt_s)

    ef = ef_ref[:]
    mu = jnp.mean(ef, axis=1, keepdims=True)
    xc = ef - mu
    var = jnp.mean(xc * xc, axis=1, keepdims=True)
    h = xc * lax.rsqrt(var + 1e-5) * lng_ref[:] + lnb_ref[:]
    h = jnp.maximum(jnp.dot(h, f1w_ref[:],
                            preferred_element_type=jnp.float32) + f1b_ref[:],
                    0.0)

    w1m = jnp.dot(h, w1w_ref[:], preferred_element_type=jnp.float32) + w1b_ref[:]
    w2m = jnp.dot(h, w2w_ref[:], preferred_element_type=jnp.float32) + w2b_ref[:]
    w3m = jnp.dot(h, w3w_ref[:], preferred_element_type=jnp.float32) + w3b_ref[:]
    w4m = jnp.dot(h, w4w_ref[:], preferred_element_type=jnp.float32) + w4b_ref[:]
    w5m = jnp.dot(h, w5w_ref[:], preferred_element_type=jnp.float32) + w5b_ref[:]

    dstv = dst_ref[0]
    iota_n = lax.broadcasted_iota(jnp.int32, (BE, L), 1)
    ohd = (dstv == iota_n).astype(jnp.float32)
    xs = jnp.dot(ohd, l0_s[:], preferred_element_type=jnp.float32)
    xv = jnp.dot(ohd, l1f_ref[:], preferred_element_type=jnp.float32)

    sh = sh_ref[:]
    x2s = sh[:, 0:1]
    x2v = sh[:, 1:4]
    x2t = sh[:, 4:9]

    def md(a, b):
        return jnp.dot(a, b, preferred_element_type=jnp.float32)

    xs2 = xs * x2s
    o0 = _N0 * md(w1m * md(xs2, e32w_ref[:]), f1024_ref[:])
    x2v24 = md(x2v, rm24_ref[:])
    dotp = md(xv * x2v24, rf248_ref[:])
    o0 = o0 + (_N0 * _I3) * md(w4m * md(dotp, e832_ref[:]), f25632_ref[:])
    t2 = md(w2m * md(xs, e328_ref[:]), f2568_ref[:])
    out1 = (_N1 * _I3) * (md(t2, rv24_ref[:]) * x2v24)
    xvs = xv * x2s
    out1 = out1 + (_N1 * _I3) * md(
        md(w3m, ra64_ref[:]) * md(xvs, rb24_ref[:]), rf192_ref[:])
    dmat = md(x2t, cd_ref[:])
    tmp = md(md(xv, ra24_ref[:]) * md(dmat, rb9_ref[:]), rf72_ref[:])
    out1 = out1 + _N1 * md(
        md(w5m, ra64_ref[:]) * md(tmp, rb24_ref[:]), rf192_ref[:])

    srow = src_ref[0]
    iota_t = lax.broadcasted_iota(jnp.int32, (L, BE), 0)
    ohsT = (srow == iota_t).astype(jnp.float32)
    acc0_s[:] = acc0_s[:] + md(ohsT, o0)
    acc1_s[:] = acc1_s[:] + md(ohsT, out1)
    cnt_s[:] = cnt_s[:] + md(ohsT, jnp.ones((BE, 1), jnp.float32))

    @pl.when(g == G - 1)
    def _finish():
        cnt = jnp.maximum(cnt_s[:], 1.0)
        m0 = acc0_s[:] / cnt
        m1 = acc1_s[:] / cnt
        nodeout_ref[:] = (jnp.dot(m0, pnw_ref[:],
                                  preferred_element_type=jnp.float32)
                          + pnb_ref[:] + node_ref[:])
        l1o_ref[:] = m1 + l1f_ref[:]


def _rep(shape):
    nd = len(shape)
    return pl.BlockSpec(shape, lambda g, _n=nd: (0,) * _n)


def _build_tc(interpret: bool = False):
    in_specs = [
        pl.BlockSpec((BE, D_PAIR), lambda g: (g, 0)),
        pl.BlockSpec((1, 1, BE), lambda g: (g, 0, 0)),
        pl.BlockSpec((1, BE, 1), lambda g: (g, 0, 0)),
        pl.BlockSpec((BE, 9), lambda g: (g, 0)),
        _rep((L, D_NODE)),
        _rep((L, 3 * L1)),
        _rep((D_NODE, L0)), _rep((1, L0)),
        _rep((1, D_PAIR)), _rep((1, D_PAIR)),
        _rep((D_PAIR, D_PAIR)), _rep((1, D_PAIR)),
        _rep((D_PAIR, 1024)), _rep((1, 1024)),
        _rep((D_PAIR, 256)), _rep((1, 256)),
        _rep((D_PAIR, 64)), _rep((1, 64)),
        _rep((D_PAIR, 256)), _rep((1, 256)),
        _rep((D_PAIR, 64)), _rep((1, 64)),
        _rep((L0, D_NODE)), _rep((1, D_NODE)),
        _rep((5, 9)),
        _rep((32, 1024)), _rep((1024, 32)),
        _rep((32, 256)), _rep((256, 8)),
        _rep((8, 256)), _rep((256, 32)),
        _rep((3, 24)), _rep((24, 8)), _rep((8, 24)),
        _rep((64, 192)), _rep((24, 192)), _rep((192, 24)),
        _rep((24, 72)), _rep((9, 72)), _rep((72, 24)),
    ]
    out_specs = [
        pl.BlockSpec((L, D_NODE), lambda g: (0, 0)),
        pl.BlockSpec((L, 3 * L1), lambda g: (0, 0)),
    ]
    return pl.pallas_call(
        _tc_body,
        grid=(G,),
        in_specs=in_specs,
        out_specs=out_specs,
        out_shape=[
            jax.ShapeDtypeStruct((L, D_NODE), jnp.float32),
            jax.ShapeDtypeStruct((L, 3 * L1), jnp.float32),
        ],
        scratch_shapes=[
            pltpu.VMEM((L, L0), jnp.float32),
            pltpu.VMEM((L, L0), jnp.float32),
            pltpu.VMEM((L, 3 * L1), jnp.float32),
            pltpu.VMEM((L, 1), jnp.float32),
        ],
        compiler_params=pltpu.CompilerParams(
            dimension_semantics=("arbitrary",)),
        interpret=interpret,
    )


def kernel(node, pair, l1_feats, pair_index, edge_src, edge_dst, edge_sh,
           proj_l0_w, proj_l0_b, ln_g, ln_b, fc1_w, fc1_b, fc2_w, fc2_b,
           proj_node_w, proj_node_b):
    table = pair.reshape(L * L, D_PAIR)
    ef = _build_sc_gather()(pair_index[1], pair_index[2], table)

    r1 = lambda a: a.reshape(1, -1)
    args = (
        ef,
        edge_src.reshape(G, 1, BE),
        edge_dst.reshape(G, BE, 1),
        edge_sh,
        node.reshape(L, D_NODE),
        l1_feats.reshape(L, 3 * L1),
        proj_l0_w, r1(proj_l0_b), r1(ln_g), r1(ln_b),
        fc1_w, r1(fc1_b),
        fc2_w[:, 0:1024], r1(fc2_b[0:1024]),
        fc2_w[:, 1024:1280], r1(fc2_b[1024:1280]),
        fc2_w[:, 1280:1344], r1(fc2_b[1280:1344]),
        fc2_w[:, 1344:1600], r1(fc2_b[1344:1600]),
        fc2_w[:, 1600:1664], r1(fc2_b[1600:1664]),
        proj_node_w, r1(proj_node_b),
        _CD,
        _R_EXP_32_1024, _R_FOLD_1024_32,
        _R_EXP_32_256, _R_FOLD_256_8,
        _R_EXP_8_256, _R_FOLD_256_32,
        _R_M_24, _R_FOLD_24_8, _R_V_24,
        _R_A_64_192, _R_B_24_192, _R_F_192_24,
        _R_A_24_72, _R_B_9_72, _R_F_72_24,
    )
    node_out, l1o = _build_tc()(*args)
    return (node_out.reshape(1, L, D_NODE), l1o.reshape(1, L, 3 * L1))


# speedup vs baseline: 4.7792x; 1.0622x over previous
import functools

import jax
import jax.numpy as jnp
import numpy as np
from jax import lax
from jax.experimental import pallas as pl
from jax.experimental.pallas import tpu as pltpu
from jax.experimental.pallas import tpu_sc as plsc

L = 512
E = 16384
D_NODE = 256
D_PAIR = 128
L0 = 32
L1 = 8
BE = 2048
G = E // BE
NW = 32
EPW = E // NW
GC = 4
CW = EPW // GC

_N0 = 1.0 / np.sqrt(40.0)
_N1 = np.sqrt(3.0 / 48.0)
_I3 = 1.0 / np.sqrt(3.0)


def _np_c121():
    C = np.zeros((3, 3, 5), dtype=np.float32)
    c = 1.0 / np.sqrt(10.0)
    d = 1.0 / np.sqrt(30.0)
    C[0, 2, 0] = c; C[2, 0, 0] = c
    C[0, 1, 1] = c; C[1, 0, 1] = c
    C[1, 1, 2] = 2.0 * d; C[0, 0, 2] = -d; C[2, 2, 2] = -d
    C[1, 2, 3] = c; C[2, 1, 3] = c
    C[2, 2, 4] = c; C[0, 0, 4] = -c
    return np.transpose(C, (2, 0, 1)).reshape(5, 9)


def _f32(x):
    return jnp.asarray(x, dtype=jnp.float32)


_E = np.eye
_O = np.ones
_R_EXP_32_1024 = _f32(np.kron(_E(32), _O((1, 32))))
_R_FOLD_1024_32 = _f32(np.tile(_E(32), (32, 1)))
_R_EXP_32_256 = _f32(np.kron(_E(32), _O((1, 8))))
_R_FOLD_256_8 = _f32(np.tile(_E(8), (32, 1)))
_R_EXP_8_256 = _f32(np.kron(_E(8), _O((1, 32))))
_R_FOLD_256_32 = _f32(np.tile(_E(32), (8, 1)))
_R_M_24 = _f32(np.tile(_E(3), (1, 8)))
_R_FOLD_24_8 = _f32(np.kron(_E(8), _O((3, 1))))
_R_V_24 = _f32(np.kron(_E(8), _O((1, 3))))
_R_A_64_192 = _f32(np.kron(_E(64), _O((1, 3))))
_R_B_24_192 = _f32(np.kron(_E(8), np.tile(_E(3), (1, 8))))
_R_F_192_24 = _f32(np.tile(_E(24), (8, 1)))
_R_A_24_72 = _f32(np.kron(_E(24), _O((1, 3))))
_R_B_9_72 = _f32(np.tile(_E(9), (1, 8)))
_R_F_72_24 = _f32(np.kron(_E(8), np.tile(_E(3), (3, 1))))
_CD = _f32(_np_c121())


@functools.cache
def _build_sc_gather():
    @functools.partial(
        pl.kernel,
        mesh=plsc.VectorSubcoreMesh(core_axis_name="c", subcore_axis_name="s"),
        out_type=jax.ShapeDtypeStruct((E, D_PAIR), jnp.float32),
        scratch_types=[
            pltpu.VMEM((EPW,), jnp.int32),
            pltpu.VMEM((EPW,), jnp.int32),
            pltpu.VMEM((GC, CW), jnp.int32),
            pltpu.VMEM((GC, CW, D_PAIR), jnp.float32),
            pltpu.SemaphoreType.DMA,
        ],
    )
    def _sc_gather(ii_hbm, jj_hbm, table_hbm, out_hbm, ii_v, jj_v, idx_v,
                   rows_v, sem):
        wid = lax.axis_index("s") * 2 + lax.axis_index("c")
        base = wid * EPW
        pltpu.sync_copy(ii_hbm.at[pl.ds(base, EPW)], ii_v)
        pltpu.sync_copy(jj_hbm.at[pl.ds(base, EPW)], jj_v)
        for c in range(GC):
            for k in range(CW // 16):
                s = pl.ds(c * CW + k * 16, 16)
                idx_v[c, pl.ds(k * 16, 16)] = ii_v[s] * L + jj_v[s]
        copies = [
            pltpu.async_copy(table_hbm.at[idx_v.at[c]], rows_v.at[c], sem)
            for c in range(GC)
        ]
        for cp in copies:
            cp.wait()
        for c in range(GC):
            pltpu.sync_copy(rows_v.at[c],
                            out_hbm.at[pl.ds(base + c * CW, CW)])

    return _sc_gather


def _tc_body(ef_ref, src_ref, dst_ref, sh_ref, node_ref, l1f_ref,
             plw_ref, plb_ref, lng_ref, lnb_ref, f1w_ref, f1b_ref,
             wmw_ref, wmb_ref, pnw_ref, pnb_ref,
             cd_ref, e32w_ref, f1024_ref, e328_ref, f2568_ref, e832_ref,
             f25632_ref, rm24_ref, rf248_ref, rv24_ref, ra64_ref, rb24_ref,
             rf192_ref, ra24_ref, rb9_ref, rf72_ref,
             nodeout_ref, l1o_ref,
             ng_s, acc0_s, acc1_s, cnt_s):
    g = pl.program_id(0)

    @pl.when(g == 0)
    def _init():
        ng_s[:, 0:L0] = jnp.dot(node_ref[:], plw_ref[:],
                                preferred_element_type=jnp.float32) + plb_ref[:]
        ng_s[:, L0:L0 + 3 * L1] = l1f_ref[:]
        acc0_s[:] = jnp.zeros_like(acc0_s)
        acc1_s[:] = jnp.zeros_like(acc1_s)
        cnt_s[:] = jnp.zeros_like(cnt_s)

    ef = ef_ref[:]
    mu = jnp.mean(ef, axis=1, keepdims=True)
    xc = ef - mu
    var = jnp.mean(xc * xc, axis=1, keepdims=True)
    h = xc * lax.rsqrt(var + 1e-5) * lng_ref[:] + lnb_ref[:]
    h = jnp.maximum(jnp.dot(h, f1w_ref[:],
                            preferred_element_type=jnp.float32) + f1b_ref[:],
                    0.0)

    wm = jnp.dot(h, wmw_ref[:], preferred_element_type=jnp.float32) + wmb_ref[:]
    w1m = wm[:, 0:1024]
    w2m = wm[:, 1024:1280]
    w3m = wm[:, 1280:1344]
    w4m = wm[:, 1408:1664]
    w5m = wm[:, 1664:1728]

    dstv = dst_ref[0]
    iota_n = lax.broadcasted_iota(jnp.int32, (BE, L), 1)
    ohd = (dstv == iota_n).astype(jnp.float32)
    xg = jnp.dot(ohd, ng_s[:], preferred_element_type=jnp.float32)
    xs = xg[:, 0:L0]
    xv = xg[:, L0:L0 + 3 * L1]

    sh = sh_ref[:]
    x2s = sh[:, 0:1]
    x2v = sh[:, 1:4]
    x2t = sh[:, 4:9]

    def md(a, b):
        return jnp.dot(a, b, preferred_element_type=jnp.float32)

    xs2 = xs * x2s
    o0 = _N0 * md(w1m * md(xs2, e32w_ref[:]), f1024_ref[:])
    x2v24 = md(x2v, rm24_ref[:])
    dotp = md(xv * x2v24, rf248_ref[:])
    o0 = o0 + (_N0 * _I3) * md(w4m * md(dotp, e832_ref[:]), f25632_ref[:])
    t2 = md(w2m * md(xs, e328_ref[:]), f2568_ref[:])
    out1 = (_N1 * _I3) * (md(t2, rv24_ref[:]) * x2v24)
    xvs = xv * x2s
    out1 = out1 + (_N1 * _I3) * md(
        md(w3m, ra64_ref[:]) * md(xvs, rb24_ref[:]), rf192_ref[:])
    dmat = md(x2t, cd_ref[:])
    tmp = md(md(xv, ra24_ref[:]) * md(dmat, rb9_ref[:]), rf72_ref[:])
    out1 = out1 + _N1 * md(
        md(w5m, ra64_ref[:]) * md(tmp, rb24_ref[:]), rf192_ref[:])

    srow = src_ref[0]
    iota_t = lax.broadcasted_iota(jnp.int32, (L, BE), 0)
    ohsT = (srow == iota_t).astype(jnp.float32)
    acc0_s[:] = acc0_s[:] + md(ohsT, o0)
    acc1_s[:] = acc1_s[:] + md(ohsT, out1)
    cnt_s[:] = cnt_s[:] + jnp.sum(ohsT, axis=1, keepdims=True)

    @pl.when(g == G - 1)
    def _finish():
        cnt = jnp.maximum(cnt_s[:], 1.0)
        m0 = acc0_s[:] / cnt
        m1 = acc1_s[:] / cnt
        nodeout_ref[:] = (jnp.dot(m0, pnw_ref[:],
                                  preferred_element_type=jnp.float32)
                          + pnb_ref[:] + node_ref[:])
        l1o_ref[:] = m1 + l1f_ref[:]


def _rep(shape):
    nd = len(shape)
    return pl.BlockSpec(shape, lambda g, _n=nd: (0,) * _n)


def _build_tc(interpret: bool = False):
    in_specs = [
        pl.BlockSpec((BE, D_PAIR), lambda g: (g, 0)),
        pl.BlockSpec((1, 1, BE), lambda g: (g, 0, 0)),
        pl.BlockSpec((1, BE, 1), lambda g: (g, 0, 0)),
        pl.BlockSpec((BE, 9), lambda g: (g, 0)),
        _rep((L, D_NODE)),
        _rep((L, 3 * L1)),
        _rep((D_NODE, L0)), _rep((1, L0)),
        _rep((1, D_PAIR)), _rep((1, D_PAIR)),
        _rep((D_PAIR, D_PAIR)), _rep((1, D_PAIR)),
        _rep((D_PAIR, 1792)), _rep((1, 1792)),
        _rep((L0, D_NODE)), _rep((1, D_NODE)),
        _rep((5, 9)),
        _rep((32, 1024)), _rep((1024, 32)),
        _rep((32, 256)), _rep((256, 8)),
        _rep((8, 256)), _rep((256, 32)),
        _rep((3, 24)), _rep((24, 8)), _rep((8, 24)),
        _rep((64, 192)), _rep((24, 192)), _rep((192, 24)),
        _rep((24, 72)), _rep((9, 72)), _rep((72, 24)),
    ]
    out_specs = [
        pl.BlockSpec((L, D_NODE), lambda g: (0, 0)),
        pl.BlockSpec((L, 3 * L1), lambda g: (0, 0)),
    ]
    return pl.pallas_call(
        _tc_body,
        grid=(G,),
        in_specs=in_specs,
        out_specs=out_specs,
        out_shape=[
            jax.ShapeDtypeStruct((L, D_NODE), jnp.float32),
            jax.ShapeDtypeStruct((L, 3 * L1), jnp.float32),
        ],
        scratch_shapes=[
            pltpu.VMEM((L, L0 + 3 * L1), jnp.float32),
            pltpu.VMEM((L, L0), jnp.float32),
            pltpu.VMEM((L, 3 * L1), jnp.float32),
            pltpu.VMEM((L, 1), jnp.float32),
        ],
        compiler_params=pltpu.CompilerParams(
            dimension_semantics=("arbitrary",)),
        interpret=interpret,
    )


def _pad_fc2(w):
    z = jnp.zeros((w.shape[0], 64), jnp.float32)
    return jnp.concatenate(
        [w[:, 0:1280], w[:, 1280:1344], z, w[:, 1344:1600],
         w[:, 1600:1664], z], axis=1)


def kernel(node, pair, l1_feats, pair_index, edge_src, edge_dst, edge_sh,
           proj_l0_w, proj_l0_b, ln_g, ln_b, fc1_w, fc1_b, fc2_w, fc2_b,
           proj_node_w, proj_node_b):
    table = pair.reshape(L * L, D_PAIR)
    ef = _build_sc_gather()(pair_index[1], pair_index[2], table)

    r1 = lambda a: a.reshape(1, -1)
    args = (
        ef,
        edge_src.reshape(G, 1, BE),
        edge_dst.reshape(G, BE, 1),
        edge_sh,
        node.reshape(L, D_NODE),
        l1_feats.reshape(L, 3 * L1),
        proj_l0_w, r1(proj_l0_b), r1(ln_g), r1(ln_b),
        fc1_w, r1(fc1_b),
        _pad_fc2(fc2_w), _pad_fc2(fc2_b.reshape(1, -1)),
        proj_node_w, r1(proj_node_b),
        _CD,
        _R_EXP_32_1024, _R_FOLD_1024_32,
        _R_EXP_32_256, _R_FOLD_256_8,
        _R_EXP_8_256, _R_FOLD_256_32,
        _R_M_24, _R_FOLD_24_8, _R_V_24,
        _R_A_64_192, _R_B_24_192, _R_F_192_24,
        _R_A_24_72, _R_B_9_72, _R_F_72_24,
    )
    node_out, l1o = _build_tc()(*args)
    return (node_out.reshape(1, L, D_NODE), l1o.reshape(1, L, 3 * L1))
